# Initial kernel scaffold; baseline (speedup 1.0000x reference)
import jax
import jax.numpy as jnp
from jax.experimental import pallas as pl


def kernel(Uold, src, dst):
    raise NotImplementedError("write your pallas kernel here")


# speedup vs baseline: 13.7708x; 13.7708x over previous
import jax
import jax.numpy as jnp
from jax import lax
from jax.experimental import pallas as pl
from jax.experimental.pallas import tpu as pltpu
from jax.experimental.pallas import tpu_sc as plsc

M = 2048
C = 128
NC, NS = 2, 16
MP = M + 16


def _sc_segsum(table, src_p, dst_p, n_chunks, n, d):
    def body(tab_hbm, src_hbm, dst_hbm, out_hbm,
             acc, src_v, dst_v, ig0, ig1, ia1, gb0, gb1, zrow):
        c = lax.axis_index("c")
        s = lax.axis_index("s")

        z = jnp.zeros((16,), jnp.float32)
        for i in range(16):
            for k in range(d // 16):
                zrow[i, pl.ds(k * 16, 16)] = z
        rows_per_tile = M // NS
        for j in range(2):
            for r in range(rows_per_tile // 16):
                pltpu.sync_copy(
                    zrow, acc.at[pl.ds(j * MP + s * rows_per_tile + r * 16, 16)]
                )

        @pl.when(s == NS - 1)
        def _zero_dummy():
            for j in range(2):
                pltpu.sync_copy(zrow, acc.at[pl.ds(j * MP + M, 16)])

        plsc.subcore_barrier()

        b0n = (2 * c) * n

        def chunk(g, carry):
            off = (s * n_chunks + g) * C
            pltpu.sync_copy(src_hbm.at[pl.ds(off, C)], src_v)
            pltpu.sync_copy(dst_hbm.at[pl.ds(off, C)], dst_v)
            for i in range(C // 16):
                sl = pl.ds(i * 16, 16)
                sv = src_v[sl]
                ig0[sl] = sv + b0n
                ig1[sl] = sv + (b0n + n)
                ia1[sl] = dst_v[sl] + MP
            pltpu.sync_copy(tab_hbm.at[ig0], gb0)
            pltpu.sync_copy(tab_hbm.at[ig1], gb1)
            pltpu.sync_copy(gb0, acc.at[dst_v], add=True)
            pltpu.sync_copy(gb1, acc.at[ia1], add=True)
            return carry

        lax.fori_loop(0, n_chunks, chunk, 0)

        plsc.subcore_barrier()

        for j in range(2):
            gb = gb0 if j == 0 else gb1
            pltpu.sync_copy(acc.at[pl.ds(j * MP + s * rows_per_tile, rows_per_tile)], gb)
            pltpu.sync_copy(gb, out_hbm.at[c, j, pl.ds(s * rows_per_tile, rows_per_tile)])

    fn = pl.kernel(
        body,
        out_type=jax.ShapeDtypeStruct((NC, 2, M, d), jnp.float32),
        mesh=plsc.VectorSubcoreMesh(core_axis_name="c", subcore_axis_name="s"),
        scratch_types=[
            pltpu.VMEM_SHARED((2 * MP, d), jnp.float32),
            pltpu.VMEM((C,), jnp.int32),
            pltpu.VMEM((C,), jnp.int32),
            pltpu.VMEM((C,), jnp.int32),
            pltpu.VMEM((C,), jnp.int32),
            pltpu.VMEM((C,), jnp.int32),
            pltpu.VMEM((C, d), jnp.float32),
            pltpu.VMEM((C, d), jnp.float32),
            pltpu.VMEM((16, d), jnp.float32),
        ],
    )
    return fn(table, src_p, dst_p)


def kernel(Uold, src, dst):
    b, n, d = Uold.shape
    e = src.shape[0]

    table = Uold.reshape(b * n, d)

    gran = NS * C
    e_pad = ((e + gran - 1) // gran) * gran
    n_chunks = e_pad // gran
    pad = e_pad - e
    src_p = jnp.concatenate([src, jnp.zeros((pad,), jnp.int32)])
    dst_p = jnp.concatenate([dst, jnp.full((pad,), M, jnp.int32)])

    out4 = _sc_segsum(table, src_p, dst_p, n_chunks, n, d)
    return out4.reshape(b, M, d)


# speedup vs baseline: 24.4027x; 1.7721x over previous
import jax
import jax.numpy as jnp
from jax import lax
from jax.experimental import pallas as pl
from jax.experimental.pallas import tpu as pltpu
from jax.experimental.pallas import tpu_sc as plsc

M = 2048
C = 128
NC, NS = 2, 16
MP = M + 16


def _sc_segsum(table, gidx, sidx, n_chunks, d):
    def body(tab_hbm, gidx_hbm, sidx_hbm, out_hbm,
             acc, igA, igB, isA, isB, gb0a, gb1a, gb0b, gb1b, zrow,
             semA, semB):
        c = lax.axis_index("c")
        s = lax.axis_index("s")

        z = jnp.zeros((16,), jnp.float32)
        for i in range(16):
            for k in range(d // 16):
                zrow[i, pl.ds(k * 16, 16)] = z
        rows_per_tile = M // NS
        for j in range(2):
            for r in range(rows_per_tile // 16):
                pltpu.sync_copy(
                    zrow, acc.at[pl.ds(j * MP + s * rows_per_tile + r * 16, 16)]
                )

        @pl.when(s == NS - 1)
        def _zero_dummy():
            for j in range(2):
                pltpu.sync_copy(zrow, acc.at[pl.ds(j * MP + M, 16)])

        pltpu.sync_copy(gidx_hbm.at[2 * c, s], igA)
        pltpu.sync_copy(gidx_hbm.at[2 * c + 1, s], igB)
        pltpu.sync_copy(sidx_hbm.at[0, s], isA)
        pltpu.sync_copy(sidx_hbm.at[1, s], isB)

        plsc.subcore_barrier()

        def start_gathers(g, gb0, gb1, sem):
            pltpu.async_copy(tab_hbm.at[igA.at[g]], gb0, sem)
            pltpu.async_copy(tab_hbm.at[igB.at[g]], gb1, sem)

        def finish_chunk(g, gb0, gb1, sem):
            pltpu.make_async_copy(tab_hbm.at[igA.at[g]], gb0, sem).wait()
            pltpu.make_async_copy(tab_hbm.at[igB.at[g]], gb1, sem).wait()
            pltpu.sync_copy(gb0, acc.at[isA.at[g]], add=True)
            pltpu.sync_copy(gb1, acc.at[isB.at[g]], add=True)

        start_gathers(0, gb0a, gb1a, semA)

        def outer(t, carry):
            g0 = 2 * t
            start_gathers(g0 + 1, gb0b, gb1b, semB)
            finish_chunk(g0, gb0a, gb1a, semA)

            @pl.when(g0 + 2 < n_chunks)
            def _prefetch():
                start_gathers(g0 + 2, gb0a, gb1a, semA)

            finish_chunk(g0 + 1, gb0b, gb1b, semB)
            return carry

        lax.fori_loop(0, n_chunks // 2, outer, 0)

        plsc.subcore_barrier()

        for j in range(2):
            gb = gb0a if j == 0 else gb1a
            pltpu.sync_copy(acc.at[pl.ds(j * MP + s * rows_per_tile, rows_per_tile)], gb)
            pltpu.sync_copy(gb, out_hbm.at[c, j, pl.ds(s * rows_per_tile, rows_per_tile)])

    fn = pl.kernel(
        body,
        out_type=jax.ShapeDtypeStruct((NC, 2, M, d), jnp.float32),
        mesh=plsc.VectorSubcoreMesh(core_axis_name="c", subcore_axis_name="s"),
        scratch_types=[
            pltpu.VMEM_SHARED((2 * MP, d), jnp.float32),
            pltpu.VMEM((n_chunks, C), jnp.int32),
            pltpu.VMEM((n_chunks, C), jnp.int32),
            pltpu.VMEM((n_chunks, C), jnp.int32),
            pltpu.VMEM((n_chunks, C), jnp.int32),
            pltpu.VMEM((C, d), jnp.float32),
            pltpu.VMEM((C, d), jnp.float32),
            pltpu.VMEM((C, d), jnp.float32),
            pltpu.VMEM((C, d), jnp.float32),
            pltpu.VMEM((16, d), jnp.float32),
            pltpu.SemaphoreType.DMA,
            pltpu.SemaphoreType.DMA,
        ],
    )
    return fn(table, gidx, sidx)


def kernel(Uold, src, dst):
    b, n, d = Uold.shape
    e = src.shape[0]

    table = Uold.reshape(b * n, d)

    gran = 2 * NS * C
    e_pad = ((e + gran - 1) // gran) * gran
    n_chunks = e_pad // (NS * C)
    pad = e_pad - e
    src_p = jnp.concatenate([src, jnp.zeros((pad,), jnp.int32)])
    dst_p = jnp.concatenate([dst, jnp.full((pad,), M, jnp.int32)])

    gidx = (src_p[None, :] + (jnp.arange(b, dtype=jnp.int32) * n)[:, None]
            ).reshape(b, NS, n_chunks, C)
    sidx = (dst_p[None, :] + (jnp.arange(2, dtype=jnp.int32) * MP)[:, None]
            ).reshape(2, NS, n_chunks, C)

    out4 = _sc_segsum(table, gidx, sidx, n_chunks, d)
    return out4.reshape(b, M, d)


# speedup vs baseline: 34.6805x; 1.4212x over previous
import jax
import jax.numpy as jnp
from jax import lax
from jax.experimental import pallas as pl
from jax.experimental.pallas import tpu as pltpu
from jax.experimental.pallas import tpu_sc as plsc

M = 2048
C = 128
NC, NS = 2, 16
MP = M + 16


def _sc_segsum(Uold, gidx, sidx, n_chunks, n, d):
    def body(u_hbm, gidx_hbm, sidx_hbm, out_hbm,
             tab, acc, ig, isA, gba, gbb, zrow, semA, semB):
        c = lax.axis_index("c")
        s = lax.axis_index("s")

        z = jnp.zeros((16,), jnp.float32)
        for i in range(16):
            for k in range(d // 16):
                zrow[i, pl.ds(k * 16, 16)] = z
        rows_per_tile = M // NS

        pltpu.sync_copy(gidx_hbm.at[s], ig)
        pltpu.sync_copy(sidx_hbm.at[s], isA)

        tab_rows_per_tile = n // NS

        for j in range(2):
            for r in range(rows_per_tile // 16):
                pltpu.sync_copy(zrow, acc.at[pl.ds(s * rows_per_tile + r * 16, 16)])

            @pl.when(s == NS - 1)
            def _zero_dummy():
                pltpu.sync_copy(zrow, acc.at[pl.ds(M, 16)])

            for r in range(tab_rows_per_tile // C):
                row0 = s * tab_rows_per_tile + r * C
                pltpu.sync_copy(u_hbm.at[2 * c + j, pl.ds(row0, C)], gba)
                pltpu.sync_copy(gba, tab.at[pl.ds(row0, C)])

            plsc.subcore_barrier()

            def start_gather(g, gb, sem):
                pltpu.async_copy(tab.at[ig.at[g]], gb, sem)

            def finish_chunk(g, gb, sem):
                pltpu.make_async_copy(tab.at[ig.at[g]], gb, sem).wait()
                pltpu.sync_copy(gb, acc.at[isA.at[g]], add=True)

            start_gather(0, gba, semA)

            def outer(t, carry):
                g0 = 2 * t
                start_gather(g0 + 1, gbb, semB)
                finish_chunk(g0, gba, semA)

                @pl.when(g0 + 2 < n_chunks)
                def _prefetch():
                    start_gather(g0 + 2, gba, semA)

                finish_chunk(g0 + 1, gbb, semB)
                return carry

            lax.fori_loop(0, n_chunks // 2, outer, 0)

            plsc.subcore_barrier()

            pltpu.sync_copy(acc.at[pl.ds(s * rows_per_tile, rows_per_tile)], gbb)
            pltpu.sync_copy(gbb, out_hbm.at[c, j, pl.ds(s * rows_per_tile, rows_per_tile)])

    fn = pl.kernel(
        body,
        out_type=jax.ShapeDtypeStruct((NC, 2, M, d), jnp.float32),
        mesh=plsc.VectorSubcoreMesh(core_axis_name="c", subcore_axis_name="s"),
        scratch_types=[
            pltpu.VMEM_SHARED((n, d), jnp.float32),
            pltpu.VMEM_SHARED((MP, d), jnp.float32),
            pltpu.VMEM((n_chunks, C), jnp.int32),
            pltpu.VMEM((n_chunks, C), jnp.int32),
            pltpu.VMEM((C, d), jnp.float32),
            pltpu.VMEM((C, d), jnp.float32),
            pltpu.VMEM((16, d), jnp.float32),
            pltpu.SemaphoreType.DMA,
            pltpu.SemaphoreType.DMA,
        ],
    )
    return fn(Uold, gidx, sidx)


def kernel(Uold, src, dst):
    b, n, d = Uold.shape
    e = src.shape[0]

    gran = 2 * NS * C
    e_pad = ((e + gran - 1) // gran) * gran
    n_chunks = e_pad // (NS * C)
    pad = e_pad - e
    src_p = jnp.concatenate([src, jnp.zeros((pad,), jnp.int32)])
    dst_p = jnp.concatenate([dst, jnp.full((pad,), M, jnp.int32)])

    gidx = src_p.reshape(NS, n_chunks, C)
    sidx = dst_p.reshape(NS, n_chunks, C)

    out4 = _sc_segsum(Uold, gidx, sidx, n_chunks, n, d)
    return out4.reshape(b, M, d)
